# Initial kernel scaffold; baseline (speedup 1.0000x reference)
import jax
import jax.numpy as jnp
from jax.experimental import pallas as pl


def kernel(f00, f01, f02, f03, f04, f05, f06, f07, f08, f09, f10, f11, f12, f13, f14, f15, f16, f17, f18, f19, f20, f21, f22, f23, f24, f25, proj0, proj1, w0, w1):
    raise NotImplementedError("write your pallas kernel here")


# speedup vs baseline: 44.2277x; 44.2277x over previous
import numpy as np
import jax
import jax.numpy as jnp
from jax.experimental import pallas as pl
from jax.experimental.pallas import tpu as pltpu

B = 16384
N_FEAT = 26
EMB = 64
QUANTILES = np.array([-1.2816, -0.8416, -0.5244, -0.2533, 0.0,
                      0.2533, 0.5244, 0.8416, 1.2816], dtype=np.float32)
NP0, NB0 = 16, 20
NP1, NB1 = 32, 50
C0 = NP0 * (NB0 + 1)
C1 = NP1 * (NB1 + 1)

NEG = np.float32(-3.0e38)


def _grid_pts(nb):
    res = 2.0 / nb
    return (np.linspace(-1.0, 1.0, nb + 1)[:-1] + 0.5 * res).astype(np.float32)


_TH0 = np.repeat(np.concatenate([[NEG], _grid_pts(NB0)]).astype(np.float32), NP0)
_TH1 = np.repeat(np.concatenate([[NEG], _grid_pts(NB1)]).astype(np.float32), NP1)

BB = 1024


def _body(x_ref, p_ref, th0_ref, th1_ref, w0_ref, w1_ref, o_ref):
    x = x_ref[...]
    b = jnp.zeros_like(x)
    for q in QUANTILES:
        b += (x > q).astype(jnp.float32)
    xq = b / np.float32(10.0) - np.float32(0.5)
    n = jnp.sqrt(jnp.sum(xq * xq, axis=1, keepdims=True))
    xn = xq / jnp.maximum(n, np.float32(1e-12))
    z = jnp.dot(xn, p_ref[...], preferred_element_type=jnp.float32)
    z0 = z[:, :NP0]
    z1 = z[:, NP0:]
    z0t = jnp.concatenate([z0] * (NB0 + 1), axis=1)
    z1t = jnp.concatenate([z1] * (NB1 + 1), axis=1)
    s0 = (z0t > th0_ref[...]).astype(jnp.bfloat16)
    s1 = (z1t > th1_ref[...]).astype(jnp.bfloat16)
    w0 = w0_ref[...]
    w1 = w1_ref[...]
    dw0 = w0 - jnp.concatenate(
        [jnp.zeros((NP0, EMB), jnp.float32), w0[:C0 - NP0]], axis=0)
    dw1 = w1 - jnp.concatenate(
        [jnp.zeros((NP1, EMB), jnp.float32), w1[:C1 - NP1]], axis=0)
    dw0h = dw0.astype(jnp.bfloat16)
    dw0l = (dw0 - dw0h.astype(jnp.float32)).astype(jnp.bfloat16)
    dw1h = dw1.astype(jnp.bfloat16)
    dw1l = (dw1 - dw1h.astype(jnp.float32)).astype(jnp.bfloat16)
    acc = jnp.dot(s0, dw0h, preferred_element_type=jnp.float32)
    acc += jnp.dot(s0, dw0l, preferred_element_type=jnp.float32)
    acc += jnp.dot(s1, dw1h, preferred_element_type=jnp.float32)
    acc += jnp.dot(s1, dw1l, preferred_element_type=jnp.float32)
    o_ref[...] = acc


def kernel(f00, f01, f02, f03, f04, f05, f06, f07, f08, f09, f10, f11,
           f12, f13, f14, f15, f16, f17, f18, f19, f20, f21, f22, f23,
           f24, f25, proj0, proj1, w0, w1):
    feats = [f00, f01, f02, f03, f04, f05, f06, f07, f08, f09, f10, f11,
             f12, f13, f14, f15, f16, f17, f18, f19, f20, f21, f22, f23,
             f24, f25]
    x = jnp.concatenate(feats, axis=1)
    p = jnp.concatenate([proj0, proj1], axis=1)
    w0r = w0.reshape(NP0, NB0 + 1, EMB).transpose(1, 0, 2).reshape(C0, EMB)
    w1r = w1.reshape(NP1, NB1 + 1, EMB).transpose(1, 0, 2).reshape(C1, EMB)
    th0 = jnp.asarray(_TH0)[None, :]
    th1 = jnp.asarray(_TH1)[None, :]

    grid = (B // BB,)
    out = pl.pallas_call(
        _body,
        grid=grid,
        in_specs=[
            pl.BlockSpec((BB, N_FEAT), lambda i: (i, 0)),
            pl.BlockSpec((N_FEAT, NP0 + NP1), lambda i: (0, 0)),
            pl.BlockSpec((1, C0), lambda i: (0, 0)),
            pl.BlockSpec((1, C1), lambda i: (0, 0)),
            pl.BlockSpec((C0, EMB), lambda i: (0, 0)),
            pl.BlockSpec((C1, EMB), lambda i: (0, 0)),
        ],
        out_specs=pl.BlockSpec((BB, EMB), lambda i: (i, 0)),
        out_shape=jax.ShapeDtypeStruct((B, EMB), jnp.float32),
    )(x, p, th0, th1, w0r, w1r)
    return out


# speedup vs baseline: 54.4619x; 1.2314x over previous
import numpy as np
import jax
import jax.numpy as jnp
from jax.experimental import pallas as pl
from jax.experimental.pallas import tpu as pltpu

B = 16384
N_FEAT = 26
EMB = 64
QUANTILES = np.array([-1.2816, -0.8416, -0.5244, -0.2533, 0.0,
                      0.2533, 0.5244, 0.8416, 1.2816], dtype=np.float32)
NP0, NB0 = 16, 20
NP1, NB1 = 32, 50
C0 = NP0 * (NB0 + 1)
C1 = NP1 * (NB1 + 1)
C = C0 + C1

NEG = np.float32(-3.0e38)


def _grid_pts(nb):
    res = 2.0 / nb
    return (np.linspace(-1.0, 1.0, nb + 1)[:-1] + 0.5 * res).astype(np.float32)


_TH = np.concatenate([
    np.repeat(np.concatenate([[NEG], _grid_pts(NB0)]).astype(np.float32), NP0),
    np.repeat(np.concatenate([[NEG], _grid_pts(NB1)]).astype(np.float32), NP1),
])

BB = 2048


def _body(x_ref, p_ref, th_ref, w0_ref, w1_ref, o_ref, dw_ref):
    @pl.when(pl.program_id(0) == 0)
    def _prep():
        w0 = w0_ref[...]
        w1 = w1_ref[...]
        dw0 = w0 - jnp.concatenate(
            [jnp.zeros((NP0, EMB), jnp.float32), w0[:C0 - NP0]], axis=0)
        dw1 = w1 - jnp.concatenate(
            [jnp.zeros((NP1, EMB), jnp.float32), w1[:C1 - NP1]], axis=0)
        dw = jnp.concatenate([dw0, dw1], axis=0)
        dwh = dw.astype(jnp.bfloat16)
        dwl = (dw - dwh.astype(jnp.float32)).astype(jnp.bfloat16)
        dw_ref[...] = jnp.concatenate([dwh, dwl], axis=1)

    x = x_ref[...]
    b = jnp.zeros_like(x)
    for q in QUANTILES:
        b += (x > q).astype(jnp.float32)
    xq = b / np.float32(10.0) - np.float32(0.5)
    n = jnp.sqrt(jnp.sum(xq * xq, axis=1, keepdims=True))
    xn = xq / jnp.maximum(n, np.float32(1e-12))
    z = jnp.dot(xn, p_ref[...], preferred_element_type=jnp.float32)
    z0 = z[:, :NP0]
    z1 = z[:, NP0:]
    zt = jnp.concatenate([z0] * (NB0 + 1) + [z1] * (NB1 + 1), axis=1)
    s = (zt > th_ref[...]).astype(jnp.bfloat16)
    acc2 = jnp.dot(s, dw_ref[...], preferred_element_type=jnp.float32)
    o_ref[...] = acc2[:, :EMB] + acc2[:, EMB:]


def kernel(f00, f01, f02, f03, f04, f05, f06, f07, f08, f09, f10, f11,
           f12, f13, f14, f15, f16, f17, f18, f19, f20, f21, f22, f23,
           f24, f25, proj0, proj1, w0, w1):
    feats = [f00, f01, f02, f03, f04, f05, f06, f07, f08, f09, f10, f11,
             f12, f13, f14, f15, f16, f17, f18, f19, f20, f21, f22, f23,
             f24, f25]
    x = jnp.concatenate(feats, axis=1)
    p = jnp.concatenate([proj0, proj1], axis=1)
    w0r = w0.reshape(NP0, NB0 + 1, EMB).transpose(1, 0, 2).reshape(C0, EMB)
    w1r = w1.reshape(NP1, NB1 + 1, EMB).transpose(1, 0, 2).reshape(C1, EMB)
    th = jnp.asarray(_TH)[None, :]

    out = pl.pallas_call(
        _body,
        grid=(B // BB,),
        in_specs=[
            pl.BlockSpec((BB, N_FEAT), lambda i: (i, 0)),
            pl.BlockSpec((N_FEAT, NP0 + NP1), lambda i: (0, 0)),
            pl.BlockSpec((1, C), lambda i: (0, 0)),
            pl.BlockSpec((C0, EMB), lambda i: (0, 0)),
            pl.BlockSpec((C1, EMB), lambda i: (0, 0)),
        ],
        out_specs=pl.BlockSpec((BB, EMB), lambda i: (i, 0)),
        out_shape=jax.ShapeDtypeStruct((B, EMB), jnp.float32),
        scratch_shapes=[pltpu.VMEM((C, 2 * EMB), jnp.bfloat16)],
    )(x, p, th, w0r, w1r)
    return out
